# baseline (speedup 1.0000x reference)
import functools

import jax
import jax.numpy as jnp
from jax import lax
from jax.experimental import pallas as pl
from jax.experimental.pallas import tpu as pltpu
from jax.experimental.pallas import tpu_sc as plsc


def kernel(inputs, word_table, pos_table):
    B, S = inputs.shape
    V, D = word_table.shape
    info = plsc.get_sparse_core_info()
    NC, NS, L = info.num_cores, info.num_subcores, info.num_lanes
    NW = NC * NS
    assert B % NW == 0 and D % L == 0 and (S * D) % 8 == 0
    seqs_per_w = B // NW
    assert seqs_per_w % 2 == 0

    idx_flat = inputs.reshape(B * S)
    table_bf = word_table.astype(jnp.bfloat16)
    pos_bf = pos_table.astype(jnp.bfloat16)
    L2 = 2 * L
    mesh = plsc.VectorSubcoreMesh(core_axis_name="c", subcore_axis_name="s")

    @functools.partial(
        pl.kernel,
        out_type=jax.ShapeDtypeStruct((B * S, D), jnp.bfloat16),
        mesh=mesh,
        scratch_types=[
            pltpu.VMEM((2, S), jnp.int32),
            pltpu.VMEM((2, S, D), jnp.bfloat16),
            pltpu.VMEM((S, D), jnp.bfloat16),
            pltpu.SemaphoreType.DMA,
            pltpu.SemaphoreType.DMA,
        ],
        compiler_params=pltpu.CompilerParams(use_tc_tiling_on_sc=False),
    )
    def emb_kernel(idx_hbm, table_hbm, pos_hbm, out_hbm,
                   idx_v, rows_v, pos_v, gsem, osem):
        wid = lax.axis_index("s") * NC + lax.axis_index("c")
        base = wid * seqs_per_w * S
        pltpu.sync_copy(pos_hbm, pos_v)

        def start_gather(b, slot):
            start = base + b * S
            pltpu.sync_copy(idx_hbm.at[pl.ds(start, S)], idx_v.at[slot])
            pltpu.async_copy(table_hbm.at[idx_v.at[slot]],
                             rows_v.at[slot], gsem)

        def gather_wait(slot):
            pltpu.make_async_copy(table_hbm.at[idx_v.at[slot]],
                                  rows_v.at[slot], gsem).wait()

        def out_wait(slot):
            pltpu.make_async_copy(rows_v.at[slot],
                                  out_hbm.at[pl.ds(0, S)], osem).wait()

        start_gather(0, 0)

        def pair_body(g, carry):
            for sl in range(2):
                b = g * 2 + sl
                gather_wait(sl)

                @pl.when(b + 1 < seqs_per_w)
                def _():
                    start_gather(b + 1, 1 - sl)

                def add_row(srow, c2):
                    for j in range(D // L2):
                        cs = pl.ds(j * L2, L2)
                        rows_v[sl, srow, cs] = (
                            rows_v[sl, srow, cs] + pos_v[srow, cs])
                    return c2

                lax.fori_loop(0, S, add_row, 0)

                @pl.when(b >= 2)
                def _():
                    out_wait(sl)

                start = base + b * S
                pltpu.async_copy(rows_v.at[sl],
                                 out_hbm.at[pl.ds(start, S)], osem)
            return carry

        lax.fori_loop(0, seqs_per_w // 2, pair_body, 0)
        out_wait(0)
        out_wait(1)

    out = emb_kernel(idx_flat, table_bf, pos_bf)
    return out.reshape(B, S, D).astype(jnp.float32)


# speedup vs baseline: 1.3815x; 1.3815x over previous
import functools

import jax
import jax.numpy as jnp
from jax import lax
from jax.experimental import pallas as pl
from jax.experimental.pallas import tpu as pltpu
from jax.experimental.pallas import tpu_sc as plsc


def kernel(inputs, word_table, pos_table):
    B, S = inputs.shape
    V, D = word_table.shape
    info = plsc.get_sparse_core_info()
    NC, NS, L = info.num_cores, info.num_subcores, info.num_lanes
    NW = NC * NS
    assert B % NW == 0 and D % L == 0 and (S * D) % 8 == 0
    seqs_per_w = B // NW
    assert seqs_per_w % 2 == 0

    idx_flat = inputs.reshape(B * S)
    mesh = plsc.VectorSubcoreMesh(core_axis_name="c", subcore_axis_name="s")

    @functools.partial(
        pl.kernel,
        out_type=jax.ShapeDtypeStruct((B * S, D), jnp.float32),
        mesh=mesh,
        scratch_types=[
            pltpu.VMEM((2, S), jnp.int32),
            pltpu.VMEM((2, S, D), jnp.float32),
            pltpu.VMEM((S, D), jnp.float32),
            pltpu.SemaphoreType.DMA,
            pltpu.SemaphoreType.DMA,
        ],
        compiler_params=pltpu.CompilerParams(use_tc_tiling_on_sc=False),
    )
    def emb_kernel(idx_hbm, table_hbm, pos_hbm, out_hbm,
                   idx_v, rows_v, pos_v, gsem, osem):
        wid = lax.axis_index("s") * NC + lax.axis_index("c")
        base = wid * seqs_per_w * S
        pltpu.sync_copy(pos_hbm, pos_v)

        def start_gather(b, slot):
            start = base + b * S
            pltpu.sync_copy(idx_hbm.at[pl.ds(start, S)], idx_v.at[slot])
            pltpu.async_copy(table_hbm.at[idx_v.at[slot]],
                             rows_v.at[slot], gsem)

        def gather_wait(slot):
            pltpu.make_async_copy(table_hbm.at[idx_v.at[slot]],
                                  rows_v.at[slot], gsem).wait()

        def out_wait(slot):
            pltpu.make_async_copy(rows_v.at[slot],
                                  out_hbm.at[pl.ds(0, S)], osem).wait()

        start_gather(0, 0)

        def pair_body(g, carry):
            for sl in range(2):
                b = g * 2 + sl
                gather_wait(sl)

                @pl.when(b + 1 < seqs_per_w)
                def _():
                    start_gather(b + 1, 1 - sl)

                def add_row(srow, c2):
                    for j in range(D // L):
                        cs = pl.ds(j * L, L)
                        rows_v[sl, srow, cs] = (
                            rows_v[sl, srow, cs] + pos_v[srow, cs])
                    return c2

                lax.fori_loop(0, S, add_row, 0)

                @pl.when(b >= 2)
                def _():
                    out_wait(sl)

                start = base + b * S
                pltpu.async_copy(rows_v.at[sl],
                                 out_hbm.at[pl.ds(start, S)], osem)
            return carry

        lax.fori_loop(0, seqs_per_w // 2, pair_body, 0)
        out_wait(0)
        out_wait(1)

    out = emb_kernel(idx_flat, word_table, pos_table)
    return out.reshape(B, S, D)


# speedup vs baseline: 2.3884x; 1.7288x over previous
import functools

import jax
import jax.numpy as jnp
from jax import lax
from jax.experimental import pallas as pl
from jax.experimental.pallas import tpu as pltpu
from jax.experimental.pallas import tpu_sc as plsc


def kernel(inputs, word_table, pos_table):
    B, S = inputs.shape
    V, D = word_table.shape
    info = plsc.get_sparse_core_info()
    NC, NS, L = info.num_cores, info.num_subcores, info.num_lanes
    assert D % NC == 0 and B % 128 == 0
    d_per_c = D // NC
    BR = B // 128
    ns_lo, hi = divmod(S, NS)
    ns_hi = ns_lo + 1
    SMAX = ns_hi * B
    VSL = (V // NS) // 128 * 128
    VREM = V - NS * VSL
    VREM_AL = VREM // 128 * 128
    VTAIL = VREM - VREM_AL

    idx_sm = inputs.T.reshape(S * B)
    idx_pad = jnp.pad(idx_sm, (0, SMAX * NS - S * B))
    pos_pad = jnp.pad(pos_table.reshape(S * D), (0, ns_hi * D * NS - S * D))
    tableT = word_table.T
    tail1d = word_table[V - VTAIL:, :].T.reshape(D * VTAIL) if VTAIL else None

    mesh = plsc.VectorSubcoreMesh(core_axis_name="c", subcore_axis_name="s")

    def make_part(d_lo, d_n):
      @functools.partial(
        pl.kernel,
        out_type=jax.ShapeDtypeStruct((S, NC, d_n, BR, 128), jnp.float32),
        mesh=mesh,
        scratch_types=[
            pltpu.VMEM_SHARED((V,), jnp.float32),
            pltpu.VMEM((SMAX,), jnp.int32),
            pltpu.VMEM((SMAX,), jnp.float32),
            pltpu.VMEM((ns_hi, BR, 128), jnp.float32),
            pltpu.VMEM((ns_hi * D,), jnp.float32),
            pltpu.VMEM((max(VTAIL, L),), jnp.float32),
            pltpu.SemaphoreType.DMA,
            pltpu.SemaphoreType.DMA,
        ],
        compiler_params=pltpu.CompilerParams(
            use_tc_tiling_on_sc=True, needs_layout_passes=False),
    )
      def emb_kernel(idx_hbm, tT, pos_hbm, tail_hbm, out_hbm,
                     slab, idx_v, vals, stage, pos_v, tvmem, ssem, gsem):
        c = lax.axis_index("c")
        t = lax.axis_index("s")
        s0 = jnp.where(t < hi, ns_hi * t, ns_lo * t + hi)
        ns = jnp.where(t < hi, ns_hi, ns_lo)
        pltpu.sync_copy(idx_hbm.at[pl.ds(s0 * B, SMAX)], idx_v)
        pltpu.sync_copy(pos_hbm.at[pl.ds(s0 * D, ns_hi * D)], pos_v)

        def slab_piece(di):
            for cc in range(NC):
                @pl.when(c == cc)
                def _(cc=cc):
                    d = cc * d_per_c + di
                    for tt in range(NS):
                        @pl.when(t == tt)
                        def _(tt=tt, d=d):
                            pltpu.async_copy(
                                tT.at[d, pl.ds(tt * VSL, VSL)],
                                slab.at[pl.ds(tt * VSL, VSL)], ssem)
                    if VREM_AL:
                        @pl.when(t == NS - 1)
                        def _(d=d):
                            pltpu.async_copy(
                                tT.at[d, pl.ds(NS * VSL, VREM_AL)],
                                slab.at[pl.ds(NS * VSL, VREM_AL)], ssem)
                    if VTAIL:
                        @pl.when(t == NS - 1)
                        def _(d=d):
                            pltpu.async_copy(
                                tail_hbm.at[pl.ds(d * VTAIL, VTAIL)],
                                tvmem.at[pl.ds(0, VTAIL)], ssem)

        def slab_wait():
            pltpu.make_async_copy(
                tT.at[0, pl.ds(0, VSL)],
                slab.at[pl.ds(pl.multiple_of(t * VSL, 128), VSL)],
                ssem).wait()
            if VREM_AL:
                @pl.when(t == NS - 1)
                def _():
                    pltpu.make_async_copy(
                        tT.at[0, pl.ds(0, VREM_AL)],
                        slab.at[pl.ds(NS * VSL, VREM_AL)], ssem).wait()
            if VTAIL:
                @pl.when(t == NS - 1)
                def _():
                    pltpu.make_async_copy(
                        tail_hbm.at[pl.ds(0, VTAIL)],
                        tvmem.at[pl.ds(0, VTAIL)], ssem).wait()
                    pltpu.sync_copy(tvmem.at[pl.ds(0, VTAIL)],
                                    slab.at[pl.ds(V - VTAIL, VTAIL)])

        slab_piece(d_lo)

        for dii in range(d_n):
            di = d_lo + dii
            slab_wait()
            plsc.subcore_barrier()
            pltpu.async_copy(slab.at[idx_v], vals, gsem).wait()
            plsc.subcore_barrier()

            if di + 1 < d_lo + d_n:
                slab_piece(di + 1)

            def sbody(si, c2, di=di):
                pv = plsc.load_gather(
                    pos_v,
                    [jnp.broadcast_to(si * D + c * d_per_c + di, (L,))])

                def jbody(j, c3):
                    sl = pl.ds(si * B + j * L, L)
                    jr = j // (128 // L)
                    jc = (j % (128 // L)) * L
                    stage[si, jr, pl.ds(jc, L)] = vals[sl] + pv
                    return c3

                lax.fori_loop(0, B // L, jbody, 0)
                return c2

            lax.fori_loop(0, ns, sbody, 0)

            for cc in range(NC):
                @pl.when(jnp.logical_and(c == cc, t < hi))
                def _(cc=cc, di=di):
                    pltpu.sync_copy(
                        stage.at[pl.ds(0, ns_hi)],
                        out_hbm.at[pl.ds(s0, ns_hi), cc, dii, :, :])

                @pl.when(jnp.logical_and(c == cc, t >= hi))
                def _(cc=cc, di=di):
                    pltpu.sync_copy(
                        stage.at[pl.ds(0, ns_lo)],
                        out_hbm.at[pl.ds(s0, ns_lo), cc, dii, :, :])

      return emb_kernel

    half = d_per_c // 2
    out_a = make_part(0, half)(idx_pad, tableT, pos_pad, tail1d)
    out_b = make_part(half, d_per_c - half)(idx_pad, tableT, pos_pad, tail1d)
    out5 = jnp.concatenate([out_a, out_b], axis=2)
    return jnp.transpose(out5.reshape(S, D, B), (2, 0, 1))
